# speedup vs baseline: 2.4239x; 2.4239x over previous
import jax
import jax.numpy as jnp
from jax.experimental import pallas as pl
from jax.experimental.pallas import tpu as pltpu

_BR = 512


def _body(x_ref, xi_ref, yi_ref, y_ref, out_ref):
    b = pl.program_id(0)
    r = pl.program_id(1)
    row_start = r * _BR

    blk = y_ref[0]
    n = xi_ref.shape[1]
    cols = blk.shape[1]

    row_iota = jax.lax.broadcasted_iota(jnp.int32, (_BR, n), 0) + row_start
    rowsel = (row_iota == xi_ref[0][None, :]).astype(jnp.float32)
    col_iota = jax.lax.broadcasted_iota(jnp.int32, (n, cols), 1)
    onehot = (col_iota == yi_ref[0][:, None]).astype(jnp.float32)

    add = jax.lax.dot(rowsel, onehot, preferred_element_type=jnp.float32)
    out_ref[0] = blk + x_ref[b, 0] * add


@jax.jit
def kernel(Y, X, x_idx, y_idx):
    B, H, W = Y.shape
    n = x_idx.shape[0]
    grid = (B, H // _BR)
    return pl.pallas_call(
        _body,
        grid=grid,
        in_specs=[
            pl.BlockSpec(memory_space=pltpu.SMEM),
            pl.BlockSpec((1, n), lambda b, r: (0, 0)),
            pl.BlockSpec((1, n), lambda b, r: (0, 0)),
            pl.BlockSpec((1, _BR, W), lambda b, r: (b, r, 0)),
        ],
        out_specs=pl.BlockSpec((1, _BR, W), lambda b, r: (b, r, 0)),
        out_shape=jax.ShapeDtypeStruct((B, H, W), Y.dtype),
        compiler_params=pltpu.CompilerParams(
            dimension_semantics=("parallel", "parallel"),
        ),
    )(X, x_idx.reshape(1, n), y_idx.reshape(1, n), Y)


# speedup vs baseline: 2.4874x; 1.0262x over previous
import jax
import jax.numpy as jnp
from jax.experimental import pallas as pl
from jax.experimental.pallas import tpu as pltpu

_BR = 1024


def _body(x_ref, xi_ref, yi_ref, y_ref, out_ref):
    b = pl.program_id(0)
    r = pl.program_id(1)
    row_start = r * _BR

    blk = y_ref[0]
    n = xi_ref.shape[1]
    cols = blk.shape[1]

    row_iota = jax.lax.broadcasted_iota(jnp.int32, (_BR, n), 0) + row_start
    rowsel = (row_iota == xi_ref[0][None, :]).astype(jnp.float32)
    col_iota = jax.lax.broadcasted_iota(jnp.int32, (n, cols), 1)
    onehot = (col_iota == yi_ref[0][:, None]).astype(jnp.float32)

    add = jax.lax.dot(rowsel, onehot, preferred_element_type=jnp.float32)
    out_ref[0] = blk + x_ref[b, 0] * add


@jax.jit
def kernel(Y, X, x_idx, y_idx):
    B, H, W = Y.shape
    n = x_idx.shape[0]
    grid = (B, H // _BR)
    return pl.pallas_call(
        _body,
        grid=grid,
        in_specs=[
            pl.BlockSpec(memory_space=pltpu.SMEM),
            pl.BlockSpec((1, n), lambda b, r: (0, 0)),
            pl.BlockSpec((1, n), lambda b, r: (0, 0)),
            pl.BlockSpec((1, _BR, W), lambda b, r: (b, r, 0)),
        ],
        out_specs=pl.BlockSpec((1, _BR, W), lambda b, r: (b, r, 0)),
        out_shape=jax.ShapeDtypeStruct((B, H, W), Y.dtype),
        compiler_params=pltpu.CompilerParams(
            dimension_semantics=("parallel", "parallel"),
        ),
    )(X, x_idx.reshape(1, n), y_idx.reshape(1, n), Y)
